# baseline (speedup 1.0000x reference)
import functools

import jax
import jax.numpy as jnp
import numpy as np
from jax import lax
from jax.experimental import pallas as pl
from jax.experimental.pallas import tpu as pltpu
from jax.experimental.pallas import tpu_sc as plsc

K = 40
BN_EPS = 1e-5
_BN_DIV = np.sqrt(1.0 + BN_EPS).astype(np.float32)
_NW = 32
_G = 4


def _actf(h):
    h = h / _BN_DIV
    return jnp.where(h >= 0, h, 0.2 * h)


def _pdsel_kernel(x_ref, pd_ref, meta_ref, *, N):
    xt = x_ref[0]
    g = lax.dot_general(xt, xt, (((1,), (1,)), ((), ())))
    sq = jnp.sum(xt * xt, axis=1)
    pd = 2.0 * g - sq[:, None] - sq[None, :]
    pd_ref[0] = pd
    i = lax.bitcast_convert_type(pd, jnp.int32)
    key = jnp.where(i >= 0, i, i ^ jnp.int32(0x7FFFFFFF))
    lo = jnp.min(key, axis=1, keepdims=True)
    hi = jnp.max(key, axis=1, keepdims=True)
    ones = jnp.ones((N, 1), jnp.float32)

    def body(_, lohi):
        lo, hi = lohi
        mid = (lo >> 1) + (hi >> 1) + (lo & hi & 1)
        maskf = jnp.where(key > mid, 1.0, 0.0)
        cnt = jnp.dot(maskf, ones)
        ge = cnt >= float(K)
        return jnp.where(ge, mid + 1, lo), jnp.where(ge, hi, mid)

    lo, hi = lax.fori_loop(0, 32, body, (lo, hi))
    t = lo
    cnt_gt = jnp.dot(jnp.where(key > t, 1.0, 0.0), ones).astype(jnp.int32)
    need = K - cnt_gt
    meta_ref[0] = jnp.concatenate(
        [jnp.broadcast_to(t, (N, 16)), jnp.broadcast_to(need, (N, 16))], axis=1)


def _pdsel(x_bnc):
    B, N, C = x_bnc.shape
    return pl.pallas_call(
        functools.partial(_pdsel_kernel, N=N),
        grid=(B,),
        in_specs=[pl.BlockSpec((1, N, C), lambda b: (b, 0, 0))],
        out_specs=[pl.BlockSpec((1, N, N), lambda b: (b, 0, 0)),
                   pl.BlockSpec((1, N, 32), lambda b: (b, 0, 0))],
        out_shape=[jax.ShapeDtypeStruct((B, N, N), jnp.float32),
                   jax.ShapeDtypeStruct((B, N, 32), jnp.int32)],
    )(x_bnc)


def _sc_edges(pd, meta, xtab, C, xcsh=None, G=4):
    BN, N = pd.shape
    TW = xtab.shape[1]
    layer1 = xcsh is not None
    EW = 128 if layer1 else 2 * C
    rows_per = BN // _NW
    ngrp = rows_per // G
    assert rows_per % G == 0 and G % 2 == 0
    nch = C // 16
    mesh = plsc.VectorSubcoreMesh(core_axis_name="c", subcore_axis_name="s")

    scratch = [
        pltpu.VMEM((2, G, N), jnp.float32),
        pltpu.VMEM((2, G, 32), jnp.int32),
        pltpu.VMEM((3, G, TW), jnp.float32),
        pltpu.VMEM((2, G, 64), jnp.int32),
        pltpu.VMEM((2, G, K, TW), jnp.float32),
        pltpu.VMEM((2, K, EW), jnp.float32),
        pltpu.SemaphoreType.DMA,
        pltpu.SemaphoreType.DMA,
        pltpu.SemaphoreType.DMA,
    ]
    if layer1:
        scratch.insert(3, pltpu.VMEM((3, G, 16), jnp.float32))

    def body(*refs):
        if layer1:
            (pd_hbm, meta_hbm, xtab_hbm, xcsh_hbm, e_hbm,
             pd_v, meta_v, xc_v, xcsh_v, idxg, rows_v, e_v,
             sem_in, sem_g, sem_out) = refs
        else:
            (pd_hbm, meta_hbm, xtab_hbm, e_hbm,
             pd_v, meta_v, xc_v, idxg, rows_v, e_v,
             sem_in, sem_g, sem_out) = refs
        wid = lax.axis_index("s") * 2 + lax.axis_index("c")
        base = wid * rows_per
        lanes = lax.iota(jnp.int32, 16)
        if layer1:
            zv = jnp.zeros((16,), jnp.float32)

            def zrow(i, _):
                for sl in range(2):
                    for cc in range(EW // 16):
                        e_v[sl, i, pl.ds(cc * 16, 16)] = zv
                return 0
            lax.fori_loop(0, K, zrow, 0)

        def in_copies(g, slot, xslot):
            r0 = base + g * G
            cps = [
                pltpu.make_async_copy(pd_hbm.at[pl.ds(r0, G)],
                                      pd_v.at[slot], sem_in),
                pltpu.make_async_copy(meta_hbm.at[pl.ds(r0, G)],
                                      meta_v.at[slot], sem_in),
                pltpu.make_async_copy(xtab_hbm.at[pl.ds(r0, G)],
                                      xc_v.at[xslot], sem_in),
            ]
            if layer1:
                cps.append(pltpu.make_async_copy(xcsh_hbm.at[pl.ds(r0, G)],
                                                 xcsh_v.at[xslot], sem_in))
            return cps

        def ebuild(pr0, pslot, pxslot, guard):
            for gi in range(G):
                pltpu.make_async_copy(
                    xtab_hbm.at[idxg.at[pslot, gi, pl.ds(0, K)]],
                    rows_v.at[pslot, gi], sem_g).wait()
                es = gi % 2

                def wait_prev(gi=gi, es=es):
                    roff = pr0 + gi - 2
                    pltpu.make_async_copy(
                        e_v.at[es], e_hbm.at[pl.ds(roff * K, K)],
                        sem_out).wait()

                if gi >= 2 or guard is None:
                    wait_prev()
                else:
                    pl.when(guard)(wait_prev)

                if layer1:
                    def edge(j4, _, gi=gi, es=es):
                        for dj in range(4):
                            j = j4 * 4 + dj
                            e_v[es, j, pl.ds(0, 16)] = (
                                rows_v[pslot, gi, j, pl.ds(0, 16)]
                                - xc_v[pxslot, gi, pl.ds(0, 16)]
                                + xcsh_v[pxslot, gi, pl.ds(0, 16)])
                        return 0
                else:
                    def edge(j4, _, gi=gi, es=es):
                        for dj in range(4):
                            j = j4 * 4 + dj
                            for cc in range(nch):
                                f = rows_v[pslot, gi, j, pl.ds(cc * 16, 16)]
                                xcv = xc_v[pxslot, gi, pl.ds(cc * 16, 16)]
                                e_v[es, j, pl.ds(cc * 16, 16)] = f - xcv
                                e_v[es, j, pl.ds(C + cc * 16, 16)] = xcv
                        return 0
                lax.fori_loop(0, K // 4, edge, 0)
                pltpu.make_async_copy(
                    e_v.at[es], e_hbm.at[pl.ds((pr0 + gi) * K, K)],
                    sem_out).start()

        for cp in in_copies(0, 0, 0):
            cp.start()

        def grp(gidx, _):
            slot = lax.rem(gidx, 2)
            xslot = lax.rem(gidx, 3)
            r0 = base + gidx * G

            @pl.when(gidx + 1 < ngrp)
            def _():
                for cp in in_copies(gidx + 1, lax.rem(gidx + 1, 2),
                                    lax.rem(gidx + 1, 3)):
                    cp.start()

            for cp in in_copies(gidx, slot, xslot):
                cp.wait()
            bbase = (r0 // N) * N

            for g0 in range(0, G, 2):
                def chunk(c2, carry, g0=g0, slot=slot):
                    offs = list(carry)
                    for dc in range(2):
                        c = c2 * 2 + dc
                        for q in range(2):
                            gi = g0 + q
                            tkey = meta_v[slot, gi, pl.ds(0, 16)]
                            needv = meta_v[slot, gi, pl.ds(16, 16)]
                            off, run_eq = offs[2 * q], offs[2 * q + 1]
                            v = pd_v[slot, gi, pl.ds(c * 16, 16)]
                            ki = lax.bitcast_convert_type(v, jnp.int32)
                            ki = jnp.where(ki >= 0, ki,
                                           ki ^ jnp.int32(0x7FFFFFFF))
                            gt = ki > tkey
                            eq = ki == tkey
                            eqi = jnp.where(eq, jnp.int32(1), jnp.int32(0))
                            csum = plsc.cumsum(eqi)
                            take = jnp.logical_and(eq,
                                                   (csum + run_eq) <= needv)
                            m = jnp.logical_or(gt, take)
                            idxs = (bbase + c * 16) + lanes
                            plsc.store_compressed(
                                idxg.at[slot, gi, pl.ds(off, 16)],
                                idxs, mask=m)
                            mi = jnp.where(m, jnp.int32(1), jnp.int32(0))
                            offs[2 * q] = off + jnp.sum(mi)
                            offs[2 * q + 1] = run_eq + jnp.sum(eqi)
                    return tuple(offs)

                z = jnp.int32(0)
                lax.fori_loop(0, N // 32, chunk, (z, z, z, z))
                for q in range(2):
                    gi = g0 + q
                    pltpu.make_async_copy(
                        xtab_hbm.at[idxg.at[slot, gi, pl.ds(0, K)]],
                        rows_v.at[slot, gi], sem_g).start()

            @pl.when(gidx >= 1)
            def _():
                ebuild(r0 - G, lax.rem(gidx + 1, 2), lax.rem(gidx + 2, 3),
                       gidx >= 2)
            return 0

        lax.fori_loop(0, ngrp, grp, 0)

        last = ngrp - 1
        ebuild(base + last * G, last % 2, last % 3, None)
        for gi in (G - 2, G - 1):
            pltpu.make_async_copy(
                e_v.at[gi % 2],
                e_hbm.at[pl.ds((base + last * G + gi) * K, K)],
                sem_out).wait()

    kern = pl.kernel(
        body,
        out_type=jax.ShapeDtypeStruct((BN * K, EW), jnp.float32),
        mesh=mesh,
        scratch_types=scratch,
        compiler_params=pltpu.CompilerParams(needs_layout_passes=False),
    )
    if layer1:
        return kern(pd, meta, xtab, xcsh)
    return kern(pd, meta, xtab)


def _edge_conv_kernel(e_ref, w_ref, o_ref, *, P):
    e = e_ref[...]
    h = jnp.dot(e, w_ref[...])
    h = _actf(h)
    h = h.reshape(P, K, -1)
    o_ref[...] = jnp.max(h, axis=1)


def _edge_conv(e, w, P=128):
    BNK, EW = e.shape
    BN = BNK // K
    D = w.shape[1]
    return pl.pallas_call(
        functools.partial(_edge_conv_kernel, P=P),
        grid=(BN // P,),
        in_specs=[
            pl.BlockSpec((P * K, EW), lambda p: (p, 0)),
            pl.BlockSpec((EW, D), lambda p: (0, 0)),
        ],
        out_specs=pl.BlockSpec((P, D), lambda p: (p, 0)),
        out_shape=jax.ShapeDtypeStruct((BN, D), jnp.float32),
    )(e, w)


def _conv_pdsel_kernel(e_ref, w_ref, o_ref, pd_ref, meta_ref, xacc_ref,
                       *, P, N):
    p = pl.program_id(1)
    e = e_ref[...]
    h = _actf(jnp.dot(e, w_ref[...]))
    h = h.reshape(P, K, -1)
    xb = jnp.max(h, axis=1)
    o_ref[...] = xb
    xacc_ref[pl.ds(p * P, P), :] = xb

    @pl.when(p == N // P - 1)
    def _():
        xt = xacc_ref[...]
        g = lax.dot_general(xt, xt, (((1,), (1,)), ((), ())))
        sq = jnp.sum(xt * xt, axis=1)
        pd = 2.0 * g - sq[:, None] - sq[None, :]
        pd_ref[0] = pd
        i = lax.bitcast_convert_type(pd, jnp.int32)
        key = jnp.where(i >= 0, i, i ^ jnp.int32(0x7FFFFFFF))
        lo = jnp.min(key, axis=1, keepdims=True)
        hi = jnp.max(key, axis=1, keepdims=True)
        ones = jnp.ones((N, 1), jnp.float32)

        def sbody(_, lohi):
            lo, hi = lohi
            mid = (lo >> 1) + (hi >> 1) + (lo & hi & 1)
            maskf = jnp.where(key > mid, 1.0, 0.0)
            cnt = jnp.dot(maskf, ones)
            ge = cnt >= float(K)
            return jnp.where(ge, mid + 1, lo), jnp.where(ge, hi, mid)

        lo, hi = lax.fori_loop(0, 32, sbody, (lo, hi))
        t = lo
        cnt_gt = jnp.dot(jnp.where(key > t, 1.0, 0.0), ones).astype(jnp.int32)
        need = K - cnt_gt
        meta_ref[0] = jnp.concatenate(
            [jnp.broadcast_to(t, (N, 16)), jnp.broadcast_to(need, (N, 16))],
            axis=1)


def _conv_pdsel(e, w, B, N, P=128):
    BNK, EW = e.shape
    BN = BNK // K
    D = w.shape[1]
    return pl.pallas_call(
        functools.partial(_conv_pdsel_kernel, P=P, N=N),
        grid=(B, N // P),
        in_specs=[
            pl.BlockSpec((P * K, EW), lambda b, p: (b * (N // P) + p, 0)),
            pl.BlockSpec((EW, D), lambda b, p: (0, 0)),
        ],
        out_specs=[
            pl.BlockSpec((P, D), lambda b, p: (b * (N // P) + p, 0)),
            pl.BlockSpec((1, N, N), lambda b, p: (b, 0, 0)),
            pl.BlockSpec((1, N, 32), lambda b, p: (b, 0, 0)),
        ],
        out_shape=[
            jax.ShapeDtypeStruct((BN, D), jnp.float32),
            jax.ShapeDtypeStruct((B, N, N), jnp.float32),
            jax.ShapeDtypeStruct((B, N, 32), jnp.int32),
        ],
        scratch_shapes=[pltpu.VMEM((N, D), jnp.float32)],
    )(e, w)


def _conv5_kernel(x_ref, w_ref, g_ref):
    x = x_ref[...]
    h = _actf(jnp.dot(x, w_ref[...]))
    p1 = jnp.max(h, axis=0)
    p2 = jnp.sum(h, axis=0) * (1.0 / x.shape[0])
    g_ref[0, 0] = jnp.concatenate([p1, p2], axis=0)


def _conv5_pool(x_cat, w5, B, N):
    C = x_cat.shape[1]
    emb = w5.shape[1]
    return pl.pallas_call(
        _conv5_kernel,
        grid=(B,),
        in_specs=[
            pl.BlockSpec((N, C), lambda b: (b, 0)),
            pl.BlockSpec((C, emb), lambda b: (0, 0)),
        ],
        out_specs=pl.BlockSpec((1, 1, 2 * emb), lambda b: (b, 0, 0)),
        out_shape=jax.ShapeDtypeStruct((B, 1, 2 * emb), jnp.float32),
    )(x_cat, w5)[:, 0, :]


def _head_kernel(g_ref, l1_ref, l2_ref, b2_ref, l3_ref, b3_ref,
                 logits_ref, logx_ref):
    g = g_ref[...]
    f = _actf(jnp.dot(g, l1_ref[...]))
    f = _actf(jnp.dot(f, l2_ref[...]) + b2_ref[...][None, :])
    logits = jnp.dot(f, l3_ref[...]) + b3_ref[...][None, :]
    m = jnp.max(logits, axis=-1, keepdims=True)
    shifted = logits - m
    logx = shifted - jnp.log(jnp.sum(jnp.exp(shifted), axis=-1, keepdims=True))
    logits_ref[...] = logits
    logx_ref[...] = logx


def _head(g, l1, l2, b2, l3, b3):
    B = g.shape[0]
    nc = l3.shape[1]
    return pl.pallas_call(
        _head_kernel,
        in_specs=[pl.BlockSpec(g.shape, lambda: (0, 0)),
                  pl.BlockSpec(l1.shape, lambda: (0, 0)),
                  pl.BlockSpec(l2.shape, lambda: (0, 0)),
                  pl.BlockSpec(b2.shape, lambda: (0,)),
                  pl.BlockSpec(l3.shape, lambda: (0, 0)),
                  pl.BlockSpec(b3.shape, lambda: (0,))],
        out_specs=[pl.BlockSpec((B, nc), lambda: (0, 0)),
                   pl.BlockSpec((B, nc), lambda: (0, 0))],
        out_shape=[jax.ShapeDtypeStruct((B, nc), jnp.float32),
                   jax.ShapeDtypeStruct((B, nc), jnp.float32)],
    )(g, l1, l2, b2, l3, b3)


def _pad_tab(x_flat):
    BN, C = x_flat.shape
    if C < 128:
        return jnp.concatenate([x_flat, jnp.zeros((BN, 128 - C), jnp.float32)], 1)
    return x_flat


def kernel(x, W1, W2, W3, W4, W5, L1, L2, b2, L3, b3):
    B, _, N = x.shape
    BN = B * N
    x_flat = jnp.transpose(x, (0, 2, 1)).reshape(BN, 3)

    pd, meta = _pdsel(x_flat.reshape(B, N, 3))
    xcsh = jnp.concatenate(
        [jnp.zeros((BN, 3), jnp.float32), x_flat,
         jnp.zeros((BN, 10), jnp.float32)], 1)
    e1 = _sc_edges(pd.reshape(BN, N), meta.reshape(BN, 32),
                   _pad_tab(x_flat), 3, xcsh, G=8)
    w1p = jnp.concatenate([W1, jnp.zeros((128 - 6, W1.shape[1]), W1.dtype)], 0)

    x1 = _edge_conv(e1, w1p)
    pd, meta = _pdsel(x1.reshape(B, N, 64))
    e2 = _sc_edges(pd.reshape(BN, N), meta.reshape(BN, 32), _pad_tab(x1), 64,
                   G=8)
    x2 = _edge_conv(e2, W2)
    pd, meta = _pdsel(x2.reshape(B, N, 64))
    e3 = _sc_edges(pd.reshape(BN, N), meta.reshape(BN, 32), _pad_tab(x2), 64,
                   G=8)
    x3 = _edge_conv(e3, W3)
    pd, meta = _pdsel(x3.reshape(B, N, 128))
    e4 = _sc_edges(pd.reshape(BN, N), meta.reshape(BN, 32), x3, 128)
    x4 = _edge_conv(e4, W4)

    x_cat = jnp.concatenate([x1, x2, x3, x4], axis=1)
    g = _conv5_pool(x_cat, W5, B, N)
    return _head(g, L1, L2, b2, L3, b3)


# speedup vs baseline: 1.0107x; 1.0107x over previous
import functools

import jax
import jax.numpy as jnp
import numpy as np
from jax import lax
from jax.experimental import pallas as pl
from jax.experimental.pallas import tpu as pltpu
from jax.experimental.pallas import tpu_sc as plsc

K = 40
BN_EPS = 1e-5
_BN_DIV = np.sqrt(1.0 + BN_EPS).astype(np.float32)
_NW = 32
_G = 4


def _actf(h):
    h = h / _BN_DIV
    return jnp.where(h >= 0, h, 0.2 * h)


def _pdsel_kernel(x_ref, pd_ref, meta_ref, *, N):
    xt = x_ref[0]
    g = lax.dot_general(xt, xt, (((1,), (1,)), ((), ())))
    sq = jnp.sum(xt * xt, axis=1)
    pd = 2.0 * g - sq[:, None] - sq[None, :]
    pd_ref[0] = pd
    i = lax.bitcast_convert_type(pd, jnp.int32)
    key = jnp.where(i >= 0, i, i ^ jnp.int32(0x7FFFFFFF))
    lo = jnp.min(key, axis=1, keepdims=True)
    hi = jnp.max(key, axis=1, keepdims=True)
    ones = jnp.ones((N, 1), jnp.float32)

    def body(_, lohi):
        lo, hi = lohi
        mid = (lo >> 1) + (hi >> 1) + (lo & hi & 1)
        maskf = jnp.where(key > mid, 1.0, 0.0)
        cnt = jnp.dot(maskf, ones)
        ge = cnt >= float(K)
        return jnp.where(ge, mid + 1, lo), jnp.where(ge, hi, mid)

    lo, hi = lax.fori_loop(0, 32, body, (lo, hi))
    t = lo
    cnt_gt = jnp.dot(jnp.where(key > t, 1.0, 0.0), ones).astype(jnp.int32)
    need = K - cnt_gt
    meta_ref[0] = jnp.concatenate(
        [jnp.broadcast_to(t, (N, 16)), jnp.broadcast_to(need, (N, 16))], axis=1)


def _pdsel(x_bnc):
    B, N, C = x_bnc.shape
    return pl.pallas_call(
        functools.partial(_pdsel_kernel, N=N),
        grid=(B,),
        in_specs=[pl.BlockSpec((1, N, C), lambda b: (b, 0, 0))],
        out_specs=[pl.BlockSpec((1, N, N), lambda b: (b, 0, 0)),
                   pl.BlockSpec((1, N, 32), lambda b: (b, 0, 0))],
        out_shape=[jax.ShapeDtypeStruct((B, N, N), jnp.float32),
                   jax.ShapeDtypeStruct((B, N, 32), jnp.int32)],
    )(x_bnc)


def _sc_edges(pd, meta, xtab, C, xcsh=None, G=4):
    BN, N = pd.shape
    TW = xtab.shape[1]
    layer1 = xcsh is not None
    EW = 128 if layer1 else 2 * C
    rows_per = BN // _NW
    ngrp = rows_per // G
    assert rows_per % G == 0 and G % 2 == 0
    nch = C // 16
    mesh = plsc.VectorSubcoreMesh(core_axis_name="c", subcore_axis_name="s")

    scratch = [
        pltpu.VMEM((2, G, N), jnp.float32),
        pltpu.VMEM((2, G, 32), jnp.int32),
        pltpu.VMEM((3, G, TW), jnp.float32),
        pltpu.VMEM((2, G, 64), jnp.int32),
        pltpu.VMEM((2, G, K, TW), jnp.float32),
        pltpu.VMEM((2, K, EW), jnp.float32),
        pltpu.SemaphoreType.DMA,
        pltpu.SemaphoreType.DMA,
        pltpu.SemaphoreType.DMA,
    ]
    if layer1:
        scratch.insert(3, pltpu.VMEM((3, G, 16), jnp.float32))

    def body(*refs):
        if layer1:
            (pd_hbm, meta_hbm, xtab_hbm, xcsh_hbm, e_hbm,
             pd_v, meta_v, xc_v, xcsh_v, idxg, rows_v, e_v,
             sem_in, sem_g, sem_out) = refs
        else:
            (pd_hbm, meta_hbm, xtab_hbm, e_hbm,
             pd_v, meta_v, xc_v, idxg, rows_v, e_v,
             sem_in, sem_g, sem_out) = refs
        wid = lax.axis_index("s") * 2 + lax.axis_index("c")
        base = wid * rows_per
        lanes = lax.iota(jnp.int32, 16)
        if layer1:
            zv = jnp.zeros((16,), jnp.float32)

            def zrow(i, _):
                for sl in range(2):
                    for cc in range(EW // 16):
                        e_v[sl, i, pl.ds(cc * 16, 16)] = zv
                return 0
            lax.fori_loop(0, K, zrow, 0)

        def in_copies(g, slot, xslot):
            r0 = base + g * G
            cps = [
                pltpu.make_async_copy(pd_hbm.at[pl.ds(r0, G)],
                                      pd_v.at[slot], sem_in),
                pltpu.make_async_copy(meta_hbm.at[pl.ds(r0, G)],
                                      meta_v.at[slot], sem_in),
                pltpu.make_async_copy(xtab_hbm.at[pl.ds(r0, G)],
                                      xc_v.at[xslot], sem_in),
            ]
            if layer1:
                cps.append(pltpu.make_async_copy(xcsh_hbm.at[pl.ds(r0, G)],
                                                 xcsh_v.at[xslot], sem_in))
            return cps

        def ebuild(pr0, pslot, pxslot, guard):
            for gi in range(G):
                pltpu.make_async_copy(
                    xtab_hbm.at[idxg.at[pslot, gi, pl.ds(0, K)]],
                    rows_v.at[pslot, gi], sem_g).wait()
                es = gi % 2

                def wait_prev(gi=gi, es=es):
                    roff = pr0 + gi - 2
                    pltpu.make_async_copy(
                        e_v.at[es], e_hbm.at[pl.ds(roff * K, K)],
                        sem_out).wait()

                if gi >= 2 or guard is None:
                    wait_prev()
                else:
                    pl.when(guard)(wait_prev)

                if layer1:
                    def edge(j4, _, gi=gi, es=es):
                        for dj in range(4):
                            j = j4 * 4 + dj
                            e_v[es, j, pl.ds(0, 16)] = (
                                rows_v[pslot, gi, j, pl.ds(0, 16)]
                                - xc_v[pxslot, gi, pl.ds(0, 16)]
                                + xcsh_v[pxslot, gi, pl.ds(0, 16)])
                        return 0
                else:
                    def edge(j4, _, gi=gi, es=es):
                        for dj in range(4):
                            j = j4 * 4 + dj
                            for cc in range(nch):
                                f = rows_v[pslot, gi, j, pl.ds(cc * 16, 16)]
                                xcv = xc_v[pxslot, gi, pl.ds(cc * 16, 16)]
                                e_v[es, j, pl.ds(cc * 16, 16)] = f - xcv
                                e_v[es, j, pl.ds(C + cc * 16, 16)] = xcv
                        return 0
                lax.fori_loop(0, K // 4, edge, 0)
                pltpu.make_async_copy(
                    e_v.at[es], e_hbm.at[pl.ds((pr0 + gi) * K, K)],
                    sem_out).start()

        for cp in in_copies(0, 0, 0):
            cp.start()

        def grp(gidx, _):
            slot = lax.rem(gidx, 2)
            xslot = lax.rem(gidx, 3)
            r0 = base + gidx * G

            @pl.when(gidx + 1 < ngrp)
            def _():
                for cp in in_copies(gidx + 1, lax.rem(gidx + 1, 2),
                                    lax.rem(gidx + 1, 3)):
                    cp.start()

            for cp in in_copies(gidx, slot, xslot):
                cp.wait()
            bbase = (r0 // N) * N

            for g0 in range(0, G, 2):
                def chunk(c2, carry, g0=g0, slot=slot):
                    offs = list(carry)
                    for dc in range(2):
                        c = c2 * 2 + dc
                        for q in range(2):
                            gi = g0 + q
                            tkey = meta_v[slot, gi, pl.ds(0, 16)]
                            needv = meta_v[slot, gi, pl.ds(16, 16)]
                            off, run_eq = offs[2 * q], offs[2 * q + 1]
                            v = pd_v[slot, gi, pl.ds(c * 16, 16)]
                            ki = lax.bitcast_convert_type(v, jnp.int32)
                            ki = jnp.where(ki >= 0, ki,
                                           ki ^ jnp.int32(0x7FFFFFFF))
                            gt = ki > tkey
                            eq = ki == tkey
                            eqi = jnp.where(eq, jnp.int32(1), jnp.int32(0))
                            csum = plsc.cumsum(eqi)
                            take = jnp.logical_and(eq,
                                                   (csum + run_eq) <= needv)
                            m = jnp.logical_or(gt, take)
                            idxs = (bbase + c * 16) + lanes
                            plsc.store_compressed(
                                idxg.at[slot, gi, pl.ds(off, 16)],
                                idxs, mask=m)
                            mi = jnp.where(m, jnp.int32(1), jnp.int32(0))
                            offs[2 * q] = off + jnp.sum(mi)
                            offs[2 * q + 1] = run_eq + jnp.sum(eqi)
                    return tuple(offs)

                z = jnp.int32(0)
                lax.fori_loop(0, N // 32, chunk, (z, z, z, z))
                for q in range(2):
                    gi = g0 + q
                    pltpu.make_async_copy(
                        xtab_hbm.at[idxg.at[slot, gi, pl.ds(0, K)]],
                        rows_v.at[slot, gi], sem_g).start()

            @pl.when(gidx >= 1)
            def _():
                ebuild(r0 - G, lax.rem(gidx + 1, 2), lax.rem(gidx + 2, 3),
                       gidx >= 2)
            return 0

        lax.fori_loop(0, ngrp, grp, 0)

        last = ngrp - 1
        ebuild(base + last * G, last % 2, last % 3, None)
        for gi in (G - 2, G - 1):
            pltpu.make_async_copy(
                e_v.at[gi % 2],
                e_hbm.at[pl.ds((base + last * G + gi) * K, K)],
                sem_out).wait()

    kern = pl.kernel(
        body,
        out_type=jax.ShapeDtypeStruct((BN * K, EW), jnp.float32),
        mesh=mesh,
        scratch_types=scratch,
        compiler_params=pltpu.CompilerParams(needs_layout_passes=False),
    )
    if layer1:
        return kern(pd, meta, xtab, xcsh)
    return kern(pd, meta, xtab)


def _edge_conv_kernel(e_ref, w_ref, o_ref, *, P):
    e = e_ref[...]
    h = jnp.dot(e, w_ref[...])
    h = _actf(h)
    h = h.reshape(P, K, -1)
    o_ref[...] = jnp.max(h, axis=1)


def _edge_conv(e, w, P=128):
    BNK, EW = e.shape
    BN = BNK // K
    D = w.shape[1]
    return pl.pallas_call(
        functools.partial(_edge_conv_kernel, P=P),
        grid=(BN // P,),
        in_specs=[
            pl.BlockSpec((P * K, EW), lambda p: (p, 0)),
            pl.BlockSpec((EW, D), lambda p: (0, 0)),
        ],
        out_specs=pl.BlockSpec((P, D), lambda p: (p, 0)),
        out_shape=jax.ShapeDtypeStruct((BN, D), jnp.float32),
    )(e, w)


def _conv_pdsel_kernel(e_ref, w_ref, o_ref, pd_ref, meta_ref, xacc_ref,
                       *, P, N):
    p = pl.program_id(1)
    e = e_ref[...]
    h = _actf(jnp.dot(e, w_ref[...]))
    h = h.reshape(P, K, -1)
    xb = jnp.max(h, axis=1)
    o_ref[...] = xb
    xacc_ref[pl.ds(p * P, P), :] = xb

    @pl.when(p == N // P - 1)
    def _():
        xt = xacc_ref[...]
        g = lax.dot_general(xt, xt, (((1,), (1,)), ((), ())))
        sq = jnp.sum(xt * xt, axis=1)
        pd = 2.0 * g - sq[:, None] - sq[None, :]
        pd_ref[0] = pd
        i = lax.bitcast_convert_type(pd, jnp.int32)
        key = jnp.where(i >= 0, i, i ^ jnp.int32(0x7FFFFFFF))
        lo = jnp.min(key, axis=1, keepdims=True)
        hi = jnp.max(key, axis=1, keepdims=True)
        ones = jnp.ones((N, 1), jnp.float32)

        def sbody(_, lohi):
            lo, hi = lohi
            mid = (lo >> 1) + (hi >> 1) + (lo & hi & 1)
            maskf = jnp.where(key > mid, 1.0, 0.0)
            cnt = jnp.dot(maskf, ones)
            ge = cnt >= float(K)
            return jnp.where(ge, mid + 1, lo), jnp.where(ge, hi, mid)

        lo, hi = lax.fori_loop(0, 32, sbody, (lo, hi))
        t = lo
        cnt_gt = jnp.dot(jnp.where(key > t, 1.0, 0.0), ones).astype(jnp.int32)
        need = K - cnt_gt
        meta_ref[0] = jnp.concatenate(
            [jnp.broadcast_to(t, (N, 16)), jnp.broadcast_to(need, (N, 16))],
            axis=1)


def _conv_pdsel(e, w, B, N, P=128):
    BNK, EW = e.shape
    BN = BNK // K
    D = w.shape[1]
    return pl.pallas_call(
        functools.partial(_conv_pdsel_kernel, P=P, N=N),
        grid=(B, N // P),
        in_specs=[
            pl.BlockSpec((P * K, EW), lambda b, p: (b * (N // P) + p, 0)),
            pl.BlockSpec((EW, D), lambda b, p: (0, 0)),
        ],
        out_specs=[
            pl.BlockSpec((P, D), lambda b, p: (b * (N // P) + p, 0)),
            pl.BlockSpec((1, N, N), lambda b, p: (b, 0, 0)),
            pl.BlockSpec((1, N, 32), lambda b, p: (b, 0, 0)),
        ],
        out_shape=[
            jax.ShapeDtypeStruct((BN, D), jnp.float32),
            jax.ShapeDtypeStruct((B, N, N), jnp.float32),
            jax.ShapeDtypeStruct((B, N, 32), jnp.int32),
        ],
        scratch_shapes=[pltpu.VMEM((N, D), jnp.float32)],
    )(e, w)


def _conv5_kernel(x_ref, w_ref, g_ref):
    x = x_ref[...]
    h = _actf(jnp.dot(x, w_ref[...]))
    p1 = jnp.max(h, axis=0)
    p2 = jnp.sum(h, axis=0) * (1.0 / x.shape[0])
    g_ref[0, 0] = jnp.concatenate([p1, p2], axis=0)


def _conv5_pool(x_cat, w5, B, N):
    C = x_cat.shape[1]
    emb = w5.shape[1]
    return pl.pallas_call(
        _conv5_kernel,
        grid=(B,),
        in_specs=[
            pl.BlockSpec((N, C), lambda b: (b, 0)),
            pl.BlockSpec((C, emb), lambda b: (0, 0)),
        ],
        out_specs=pl.BlockSpec((1, 1, 2 * emb), lambda b: (b, 0, 0)),
        out_shape=jax.ShapeDtypeStruct((B, 1, 2 * emb), jnp.float32),
    )(x_cat, w5)[:, 0, :]


def _head_kernel(g_ref, l1_ref, l2_ref, b2_ref, l3_ref, b3_ref,
                 logits_ref, logx_ref):
    g = g_ref[...]
    f = _actf(jnp.dot(g, l1_ref[...]))
    f = _actf(jnp.dot(f, l2_ref[...]) + b2_ref[...][None, :])
    logits = jnp.dot(f, l3_ref[...]) + b3_ref[...][None, :]
    m = jnp.max(logits, axis=-1, keepdims=True)
    shifted = logits - m
    logx = shifted - jnp.log(jnp.sum(jnp.exp(shifted), axis=-1, keepdims=True))
    logits_ref[...] = logits
    logx_ref[...] = logx


def _head(g, l1, l2, b2, l3, b3):
    B = g.shape[0]
    nc = l3.shape[1]
    return pl.pallas_call(
        _head_kernel,
        in_specs=[pl.BlockSpec(g.shape, lambda: (0, 0)),
                  pl.BlockSpec(l1.shape, lambda: (0, 0)),
                  pl.BlockSpec(l2.shape, lambda: (0, 0)),
                  pl.BlockSpec(b2.shape, lambda: (0,)),
                  pl.BlockSpec(l3.shape, lambda: (0, 0)),
                  pl.BlockSpec(b3.shape, lambda: (0,))],
        out_specs=[pl.BlockSpec((B, nc), lambda: (0, 0)),
                   pl.BlockSpec((B, nc), lambda: (0, 0))],
        out_shape=[jax.ShapeDtypeStruct((B, nc), jnp.float32),
                   jax.ShapeDtypeStruct((B, nc), jnp.float32)],
    )(g, l1, l2, b2, l3, b3)


def _pad_tab(x_flat):
    BN, C = x_flat.shape
    if C < 128:
        return jnp.concatenate([x_flat, jnp.zeros((BN, 128 - C), jnp.float32)], 1)
    return x_flat


def kernel(x, W1, W2, W3, W4, W5, L1, L2, b2, L3, b3):
    B, _, N = x.shape
    BN = B * N
    x_flat = jnp.transpose(x, (0, 2, 1)).reshape(BN, 3)

    pd, meta = _pdsel(x_flat.reshape(B, N, 3))
    xcsh = jnp.concatenate(
        [jnp.zeros((BN, 3), jnp.float32), x_flat,
         jnp.zeros((BN, 10), jnp.float32)], 1)
    e1 = _sc_edges(pd.reshape(BN, N), meta.reshape(BN, 32),
                   _pad_tab(x_flat), 3, xcsh)
    w1p = jnp.concatenate([W1, jnp.zeros((128 - 6, W1.shape[1]), W1.dtype)], 0)

    x1 = _edge_conv(e1, w1p)
    pd, meta = _pdsel(x1.reshape(B, N, 64))
    e2 = _sc_edges(pd.reshape(BN, N), meta.reshape(BN, 32), _pad_tab(x1), 64)
    x2 = _edge_conv(e2, W2)
    pd, meta = _pdsel(x2.reshape(B, N, 64))
    e3 = _sc_edges(pd.reshape(BN, N), meta.reshape(BN, 32), _pad_tab(x2), 64)
    x3 = _edge_conv(e3, W3)
    pd, meta = _pdsel(x3.reshape(B, N, 128))
    e4 = _sc_edges(pd.reshape(BN, N), meta.reshape(BN, 32), x3, 128)
    x4 = _edge_conv(e4, W4)

    x_cat = jnp.concatenate([x1, x2, x3, x4], axis=1)
    g = _conv5_pool(x_cat, W5, B, N)
    return _head(g, L1, L2, b2, L3, b3)
